# baseline (speedup 1.0000x reference)
import jax
import jax.numpy as jnp
from jax import lax
from jax.experimental import pallas as pl
from jax.experimental.pallas import tpu as pltpu, tpu_sc as plsc

_BATCH = 200
_DMAX = 50
_DIM = 512
_LANES = 16
_NCORES = 2
_GRP = 10
_KFAST = 6
_CMAX_FAST = 0.1


def _echo_body(z_hbm, cap_hbm, inds_hbm, out_hbm, zv, idxf, idxp, capv, outv):
    wid = lax.axis_index("s") * _NCORES + lax.axis_index("c")
    d0 = wid * _LANES

    pltpu.sync_copy(z_hbm.at[:, pl.ds(d0, _LANES)], zv)
    pltpu.sync_copy(cap_hbm.at[pl.ds(d0, _LANES)], capv)

    c = 1.0 / (1.0 + jnp.exp(-capv[...]))
    fast = jnp.max(c, axis=0) < _CMAX_FAST

    def fast_path(_):
        pltpu.sync_copy(inds_hbm.at[:, pl.ds(0, 16)], idxf)
        c2 = c * c

        def group(g, csum):
            base = g * _GRP
            ivecs = [idxf[base + s, pl.ds(0, 16)] for s in range(_GRP)]
            acc_e = [None] * _GRP
            acc_o = [None] * _GRP
            for j in range(_KFAST - 1, -1, -1):
                acc = acc_o if j % 2 else acc_e
                for s in range(_GRP):
                    row = zv[ivecs[s][2 * j + 1]]
                    acc[s] = row if acc[s] is None else acc[s] * c2 + row
            for s in range(_GRP):
                tot = acc_e[s] + c * acc_o[s]
                outv[base + s] = tot
                csum = csum + tot
            return csum

        return lax.fori_loop(0, _BATCH // _GRP, group,
                             jnp.zeros((_LANES,), jnp.float32),
                             unroll=2)

    def slow_path(_):
        pltpu.sync_copy(inds_hbm, idxp)

        def sample(b, csum):
            bvec = jnp.full((16,), b, jnp.int32)
            ivecs = [
                plsc.load_gather(
                    idxp, [bvec,
                           lax.iota(jnp.int32, 16) * 2 + (32 * blk + 1)])
                for blk in range(4)
            ]
            acc = None
            for j in range(_DMAX - 1, -1, -1):
                row = zv[ivecs[j // 16][j % 16]]
                acc = row if acc is None else acc * c + row
            outv[b] = acc
            return csum + acc

        return lax.fori_loop(0, _BATCH, sample,
                             jnp.zeros((_LANES,), jnp.float32))

    csum = lax.cond(fast, fast_path, slow_path, 0)
    mean = csum * (1.0 / _BATCH)

    def finish(g, carry):
        base = g * _GRP
        for s in range(_GRP):
            b = base + s
            outv[b] = zv[b] + c * (outv[b] - mean)
        return carry

    lax.fori_loop(0, _BATCH // _GRP, finish, 0)
    pltpu.sync_copy(outv, out_hbm.at[:, pl.ds(d0, _LANES)])


@jax.jit
def _echo(z_mean, cap_param, inds):
    mesh = plsc.VectorSubcoreMesh(core_axis_name="c", subcore_axis_name="s")
    return pl.kernel(
        _echo_body,
        out_type=jax.ShapeDtypeStruct((_BATCH, _DIM), jnp.float32),
        mesh=mesh,
        compiler_params=pltpu.CompilerParams(use_tc_tiling_on_sc=False,
                                             needs_layout_passes=False,
                                             disable_bounds_checks=True,
                                             disable_semaphore_checks=True,
                                             skip_device_barrier=True),
        scratch_types=[
            pltpu.VMEM((_BATCH, _LANES), jnp.float32),
            pltpu.VMEM((_BATCH, _LANES), jnp.int32),
            pltpu.VMEM((_BATCH, 2 * _DMAX), jnp.int32),
            pltpu.VMEM((_LANES,), jnp.float32),
            pltpu.VMEM((_BATCH, _LANES), jnp.float32),
        ],
    )(z_mean, cap_param, inds)


def kernel(z_mean, cap_param, inds):
    return _echo(z_mean, cap_param, inds.reshape(_BATCH, 2 * _DMAX))


# speedup vs baseline: 1.0030x; 1.0030x over previous
import jax
import jax.numpy as jnp
from jax import lax
from jax.experimental import pallas as pl
from jax.experimental.pallas import tpu as pltpu, tpu_sc as plsc

_BATCH = 200
_DMAX = 50
_DIM = 512
_LANES = 16
_NCORES = 2
_GRP = 10
_KFAST = 6
_CMAX_FAST = 0.1


def _echo_body(z_hbm, cap_hbm, inds_hbm, out_hbm, zv, idxf, idxp, capv, outv):
    wid = lax.axis_index("s") * _NCORES + lax.axis_index("c")
    d0 = wid * _LANES

    pltpu.sync_copy(z_hbm.at[:, pl.ds(d0, _LANES)], zv)
    pltpu.sync_copy(cap_hbm.at[pl.ds(d0, _LANES)], capv)

    c = 1.0 / (1.0 + jnp.exp(-capv[...]))
    fast = jnp.max(c, axis=0) < _CMAX_FAST

    def fast_path(_):
        pltpu.sync_copy(inds_hbm.at[:, pl.ds(0, 16)], idxf)
        c2 = c * c

        def group(g, csum):
            base = g * _GRP
            ivecs = [idxf[base + s, pl.ds(0, 16)] for s in range(_GRP)]
            acc_e = [None] * _GRP
            acc_o = [None] * _GRP
            for j in range(_KFAST - 1, -1, -1):
                acc = acc_o if j % 2 else acc_e
                for s in range(_GRP):
                    row = zv[ivecs[s][2 * j + 1]]
                    acc[s] = row if acc[s] is None else acc[s] * c2 + row
            for s in range(_GRP):
                tot = acc_e[s] + c * acc_o[s]
                outv[base + s] = tot
                csum = csum + tot
            return csum

        return lax.fori_loop(0, _BATCH // _GRP, group,
                             jnp.zeros((_LANES,), jnp.float32))

    def slow_path(_):
        pltpu.sync_copy(inds_hbm, idxp)

        def sample(b, csum):
            bvec = jnp.full((16,), b, jnp.int32)
            ivecs = [
                plsc.load_gather(
                    idxp, [bvec,
                           lax.iota(jnp.int32, 16) * 2 + (32 * blk + 1)])
                for blk in range(4)
            ]
            acc = None
            for j in range(_DMAX - 1, -1, -1):
                row = zv[ivecs[j // 16][j % 16]]
                acc = row if acc is None else acc * c + row
            outv[b] = acc
            return csum + acc

        return lax.fori_loop(0, _BATCH, sample,
                             jnp.zeros((_LANES,), jnp.float32))

    csum = lax.cond(fast, fast_path, slow_path, 0)
    mean = csum * (1.0 / _BATCH)

    def finish(g, carry):
        base = g * _GRP
        for s in range(_GRP):
            b = base + s
            outv[b] = zv[b] + c * (outv[b] - mean)
        return carry

    lax.fori_loop(0, _BATCH // _GRP, finish, 0)
    pltpu.sync_copy(outv, out_hbm.at[:, pl.ds(d0, _LANES)])


@jax.jit
def _echo(z_mean, cap_param, inds):
    mesh = plsc.VectorSubcoreMesh(core_axis_name="c", subcore_axis_name="s")
    return pl.kernel(
        _echo_body,
        out_type=jax.ShapeDtypeStruct((_BATCH, _DIM), jnp.float32),
        mesh=mesh,
        compiler_params=pltpu.CompilerParams(use_tc_tiling_on_sc=False,
                                             needs_layout_passes=False,
                                             disable_bounds_checks=True,
                                             disable_semaphore_checks=True,
                                             skip_device_barrier=True),
        scratch_types=[
            pltpu.VMEM((_BATCH, _LANES), jnp.float32),
            pltpu.VMEM((_BATCH, _LANES), jnp.int32),
            pltpu.VMEM((_BATCH, 2 * _DMAX), jnp.int32),
            pltpu.VMEM((_LANES,), jnp.float32),
            pltpu.VMEM((_BATCH, _LANES), jnp.float32),
        ],
    )(z_mean, cap_param, inds)


def kernel(z_mean, cap_param, inds):
    return _echo(z_mean, cap_param, inds.reshape(_BATCH, 2 * _DMAX))


# speedup vs baseline: 1.0640x; 1.0608x over previous
import jax
import jax.numpy as jnp
from jax import lax
from jax.experimental import pallas as pl
from jax.experimental.pallas import tpu as pltpu, tpu_sc as plsc

_BATCH = 200
_DMAX = 50
_DIM = 512
_LANES = 16
_NCORES = 2
_GRP = 10
_KFAST = 4
_CMAX_FAST = 0.05


def _echo_body(z_hbm, cap_hbm, inds_hbm, out_hbm,
               zv, idxf, idxp, capv, outv, zsem, isem):
    wid = lax.axis_index("s") * _NCORES + lax.axis_index("c")
    d0 = wid * _LANES

    zcp = pltpu.async_copy(z_hbm.at[:, pl.ds(d0, _LANES)], zv, zsem)
    icp = pltpu.async_copy(inds_hbm.at[:, pl.ds(0, 16)], idxf, isem)
    pltpu.sync_copy(cap_hbm.at[pl.ds(d0, _LANES)], capv)

    c = 1.0 / (1.0 + jnp.exp(-capv[...]))
    fast = jnp.max(c, axis=0) < _CMAX_FAST
    zcp.wait()
    icp.wait()

    def fast_path(_):
        c2 = c * c

        def group(g, csum):
            base = g * _GRP
            ivecs = [idxf[base + s, pl.ds(0, 16)] for s in range(_GRP)]
            acc_e = [None] * _GRP
            acc_o = [None] * _GRP
            for j in range(_KFAST - 1, -1, -1):
                acc = acc_o if j % 2 else acc_e
                for s in range(_GRP):
                    row = zv[ivecs[s][2 * j + 1]]
                    acc[s] = row if acc[s] is None else acc[s] * c2 + row
            for s in range(_GRP):
                tot = acc_e[s] + c * acc_o[s]
                outv[base + s] = tot
                csum = csum + tot
            return csum

        return lax.fori_loop(0, _BATCH // _GRP, group,
                             jnp.zeros((_LANES,), jnp.float32))

    def slow_path(_):
        pltpu.sync_copy(inds_hbm, idxp)

        def sample(b, csum):
            bvec = jnp.full((16,), b, jnp.int32)
            ivecs = [
                plsc.load_gather(
                    idxp, [bvec,
                           lax.iota(jnp.int32, 16) * 2 + (32 * blk + 1)])
                for blk in range(4)
            ]
            acc = None
            for j in range(_DMAX - 1, -1, -1):
                row = zv[ivecs[j // 16][j % 16]]
                acc = row if acc is None else acc * c + row
            outv[b] = acc
            return csum + acc

        return lax.fori_loop(0, _BATCH, sample,
                             jnp.zeros((_LANES,), jnp.float32))

    csum = lax.cond(fast, fast_path, slow_path, 0)
    mean = csum * (1.0 / _BATCH)

    def finish(g, carry):
        base = g * _GRP
        for s in range(_GRP):
            b = base + s
            outv[b] = zv[b] + c * (outv[b] - mean)
        return carry

    lax.fori_loop(0, _BATCH // _GRP, finish, 0)
    pltpu.sync_copy(outv, out_hbm.at[:, pl.ds(d0, _LANES)])


@jax.jit
def _echo(z_mean, cap_param, inds):
    mesh = plsc.VectorSubcoreMesh(core_axis_name="c", subcore_axis_name="s")
    return pl.kernel(
        _echo_body,
        out_type=jax.ShapeDtypeStruct((_BATCH, _DIM), jnp.float32),
        mesh=mesh,
        compiler_params=pltpu.CompilerParams(use_tc_tiling_on_sc=False,
                                             needs_layout_passes=False,
                                             disable_bounds_checks=True,
                                             disable_semaphore_checks=True,
                                             skip_device_barrier=True),
        scratch_types=[
            pltpu.VMEM((_BATCH, _LANES), jnp.float32),
            pltpu.VMEM((_BATCH, _LANES), jnp.int32),
            pltpu.VMEM((_BATCH, 2 * _DMAX), jnp.int32),
            pltpu.VMEM((_LANES,), jnp.float32),
            pltpu.VMEM((_BATCH, _LANES), jnp.float32),
            pltpu.SemaphoreType.DMA,
            pltpu.SemaphoreType.DMA,
        ],
    )(z_mean, cap_param, inds)


def kernel(z_mean, cap_param, inds):
    return _echo(z_mean, cap_param, inds.reshape(_BATCH, 2 * _DMAX))
